# baseline (speedup 1.0000x reference)
import functools

import jax
import jax.numpy as jnp
from jax import lax
from jax.experimental import pallas as pl
from jax.experimental.pallas import tpu as pltpu

_BQ = 512
_BK = 1024


def _nn_loss_kernel(a_ref, b_ref, out_ref, acc_ref, *, inv_q):
    i = pl.program_id(0)
    j = pl.program_id(1)
    nq = pl.num_programs(0)
    nk = pl.num_programs(1)

    a = a_ref[...]
    b = b_ref[...]
    g = jnp.dot(
        a.astype(jnp.bfloat16),
        b.astype(jnp.bfloat16),
        preferred_element_type=jnp.float32,
    )
    b2 = jnp.sum(b * b, axis=0)
    tile_min = jnp.min(b2[None, :] - 2.0 * g, axis=1, keepdims=True)

    @pl.when(j == 0)
    def _():
        acc_ref[...] = tile_min

    @pl.when(j > 0)
    def _():
        acc_ref[...] = jnp.minimum(acc_ref[...], tile_min)

    @pl.when(j == nk - 1)
    def _():
        a2 = jnp.sum(a * a, axis=1, keepdims=True)
        d2 = a2 + acc_ref[...]
        psum = jnp.sum(jnp.sqrt(jnp.maximum(d2, 1e-12))).reshape(1, 1)
        tot = jnp.where(i == 0, psum, out_ref[...] + psum)
        out_ref[...] = jnp.where(i == nq - 1, tot * inv_q, tot)


@jax.jit
def kernel(target_embeddings, target_slice_idx, cluster_centers):
    del target_slice_idx
    q, d = target_embeddings.shape
    k = cluster_centers.shape[0]
    centers_t = cluster_centers.T

    out = pl.pallas_call(
        functools.partial(_nn_loss_kernel, inv_q=1.0 / q),
        grid=(q // _BQ, k // _BK),
        in_specs=[
            pl.BlockSpec((_BQ, d), lambda i, j: (i, 0)),
            pl.BlockSpec((d, _BK), lambda i, j: (0, j)),
        ],
        out_specs=pl.BlockSpec((1, 1), lambda i, j: (0, 0)),
        out_shape=jax.ShapeDtypeStruct((1, 1), jnp.float32),
        scratch_shapes=[pltpu.VMEM((_BQ, 1), jnp.float32)],
    )(target_embeddings, centers_t)
    return out[0, 0]


# speedup vs baseline: 1.0576x; 1.0576x over previous
import functools

import jax
import jax.numpy as jnp
from jax.experimental import pallas as pl
from jax.experimental.pallas import tpu as pltpu

_BQ = 512
_BK = 1024


def _nn_loss_kernel(am_ref, bm_ref, out_ref, acc_ref, b2_ref, *, inv_q):
    i = pl.program_id(0)
    j = pl.program_id(1)
    nq = pl.num_programs(0)
    nk = pl.num_programs(1)

    am = am_ref[...]
    bm = bm_ref[...]

    g = jnp.dot(am, bm, preferred_element_type=jnp.float32)

    @pl.when(i == 0)
    def _():
        bf = bm.astype(jnp.float32)
        b2_ref[:, pl.ds(j * _BK, _BK)] = jnp.sum(bf * bf, axis=0, keepdims=True)

    b2 = b2_ref[:, pl.ds(j * _BK, _BK)]
    tile_min = jnp.min(b2 + g, axis=1, keepdims=True)

    @pl.when(j == 0)
    def _():
        acc_ref[...] = tile_min

    @pl.when(j > 0)
    def _():
        acc_ref[...] = jnp.minimum(acc_ref[...], tile_min)

    @pl.when(j == nk - 1)
    def _():
        af = am.astype(jnp.float32)
        a2 = 0.25 * jnp.sum(af * af, axis=1, keepdims=True)
        d2 = a2 + acc_ref[...]
        psum = jnp.sum(jnp.sqrt(jnp.maximum(d2, 1e-12))).reshape(1, 1)
        tot = jnp.where(i == 0, psum, out_ref[...] + psum)
        out_ref[...] = jnp.where(i == nq - 1, tot * inv_q, tot)


@jax.jit
def kernel(target_embeddings, target_slice_idx, cluster_centers):
    del target_slice_idx
    q, d = target_embeddings.shape
    k = cluster_centers.shape[0]
    a_mm = (-2.0 * target_embeddings).astype(jnp.bfloat16)
    b_mm = cluster_centers.T.astype(jnp.bfloat16)

    out = pl.pallas_call(
        functools.partial(_nn_loss_kernel, inv_q=1.0 / q),
        grid=(q // _BQ, k // _BK),
        in_specs=[
            pl.BlockSpec((_BQ, d), lambda i, j: (i, 0)),
            pl.BlockSpec((d, _BK), lambda i, j: (0, j)),
        ],
        out_specs=pl.BlockSpec((1, 1), lambda i, j: (0, 0)),
        out_shape=jax.ShapeDtypeStruct((1, 1), jnp.float32),
        scratch_shapes=[
            pltpu.VMEM((_BQ, 1), jnp.float32),
            pltpu.VMEM((1, k), jnp.float32),
        ],
    )(a_mm, b_mm)
    return out[0, 0]


# speedup vs baseline: 1.1111x; 1.0506x over previous
import functools

import jax
import jax.numpy as jnp
from jax.experimental import pallas as pl
from jax.experimental.pallas import tpu as pltpu

_BQ = 512
_BK = 1024
_CK = 256


def _nn_loss_kernel(am_ref, bm_ref, out_ref, acc_ref, b2_ref, *, inv_q):
    i = pl.program_id(0)
    j = pl.program_id(1)
    nq = pl.num_programs(0)
    nk = pl.num_programs(1)

    am = am_ref[...]
    bm = bm_ref[...]

    @pl.when(i == 0)
    def _():
        bf = bm.astype(jnp.float32)
        b2_ref[:, pl.ds(j * _BK, _BK)] = jnp.sum(bf * bf, axis=0, keepdims=True)

    tile_min = None
    for c in range(_BK // _CK):
        g = jnp.dot(
            am, bm[:, c * _CK : (c + 1) * _CK], preferred_element_type=jnp.float32
        )
        b2 = b2_ref[:, pl.ds(j * _BK + c * _CK, _CK)]
        m = jnp.min(b2 + g, axis=1, keepdims=True)
        tile_min = m if tile_min is None else jnp.minimum(tile_min, m)

    @pl.when(j == 0)
    def _():
        acc_ref[...] = tile_min

    @pl.when(j > 0)
    def _():
        acc_ref[...] = jnp.minimum(acc_ref[...], tile_min)

    @pl.when(j == nk - 1)
    def _():
        af = am.astype(jnp.float32)
        a2 = 0.25 * jnp.sum(af * af, axis=1, keepdims=True)
        d2 = a2 + acc_ref[...]
        psum = jnp.sum(jnp.sqrt(jnp.maximum(d2, 1e-12))).reshape(1, 1)
        tot = jnp.where(i == 0, psum, out_ref[...] + psum)
        out_ref[...] = jnp.where(i == nq - 1, tot * inv_q, tot)


@jax.jit
def kernel(target_embeddings, target_slice_idx, cluster_centers):
    del target_slice_idx
    q, d = target_embeddings.shape
    k = cluster_centers.shape[0]
    a_mm = (-2.0 * target_embeddings).astype(jnp.bfloat16)
    b_mm = cluster_centers.T.astype(jnp.bfloat16)

    out = pl.pallas_call(
        functools.partial(_nn_loss_kernel, inv_q=1.0 / q),
        grid=(q // _BQ, k // _BK),
        in_specs=[
            pl.BlockSpec((_BQ, d), lambda i, j: (i, 0)),
            pl.BlockSpec((d, _BK), lambda i, j: (0, j)),
        ],
        out_specs=pl.BlockSpec((1, 1), lambda i, j: (0, 0)),
        out_shape=jax.ShapeDtypeStruct((1, 1), jnp.float32),
        scratch_shapes=[
            pltpu.VMEM((_BQ, 1), jnp.float32),
            pltpu.VMEM((1, k), jnp.float32),
        ],
    )(a_mm, b_mm)
    return out[0, 0]


# speedup vs baseline: 1.8073x; 1.6265x over previous
import functools

import jax
import jax.numpy as jnp
from jax.experimental import pallas as pl
from jax.experimental.pallas import tpu as pltpu

_BQ = 1024
_BK = 1024
_CK = 256
_BIG = 3.0e38


def _nn_loss_kernel(am_ref, bm_ref, out_ref, acc_ref, b2_ref, *, inv_q):
    i = pl.program_id(0)
    j = pl.program_id(1)
    nq = pl.num_programs(0)
    nk = pl.num_programs(1)

    am = am_ref[...]
    bm = bm_ref[...]

    @pl.when(i == 0)
    def _():
        bf = bm.astype(jnp.float32)
        b2_ref[:, pl.ds(j * _BK, _BK)] = jnp.sum(bf * bf, axis=0, keepdims=True)

    @pl.when(j == 0)
    def _():
        acc_ref[...] = jnp.full((_BQ, _CK), _BIG, jnp.float32)

    for c in range(_BK // _CK):
        g = jnp.dot(
            am, bm[:, c * _CK : (c + 1) * _CK], preferred_element_type=jnp.float32
        )
        b2 = b2_ref[:, pl.ds(j * _BK + c * _CK, _CK)]
        acc_ref[...] = jnp.minimum(acc_ref[...], b2 + g)

    @pl.when(j == nk - 1)
    def _():
        af = am.astype(jnp.float32)
        a2 = 0.25 * jnp.sum(af * af, axis=1, keepdims=True)
        d2 = a2 + jnp.min(acc_ref[...], axis=1, keepdims=True)
        psum = jnp.sum(jnp.sqrt(jnp.maximum(d2, 1e-12))).reshape(1, 1)
        tot = jnp.where(i == 0, psum, out_ref[...] + psum)
        out_ref[...] = jnp.where(i == nq - 1, tot * inv_q, tot)


@jax.jit
def kernel(target_embeddings, target_slice_idx, cluster_centers):
    del target_slice_idx
    q, d = target_embeddings.shape
    k = cluster_centers.shape[0]
    a_mm = (-2.0 * target_embeddings).astype(jnp.bfloat16)
    b_mm = cluster_centers.T.astype(jnp.bfloat16)

    out = pl.pallas_call(
        functools.partial(_nn_loss_kernel, inv_q=1.0 / q),
        grid=(q // _BQ, k // _BK),
        in_specs=[
            pl.BlockSpec((_BQ, d), lambda i, j: (i, 0)),
            pl.BlockSpec((d, _BK), lambda i, j: (0, j)),
        ],
        out_specs=pl.BlockSpec((1, 1), lambda i, j: (0, 0)),
        out_shape=jax.ShapeDtypeStruct((1, 1), jnp.float32),
        scratch_shapes=[
            pltpu.VMEM((_BQ, _CK), jnp.float32),
            pltpu.VMEM((1, k), jnp.float32),
        ],
    )(a_mm, b_mm)
    return out[0, 0]


# speedup vs baseline: 2.1826x; 1.2077x over previous
import functools

import jax
import jax.numpy as jnp
from jax.experimental import pallas as pl
from jax.experimental.pallas import tpu as pltpu

_BQ = 2048
_BK = 1024
_CK = 256
_BIG = 3.0e38


def _nn_loss_kernel(am_ref, bm_ref, out_ref, acc_ref, b2_ref, *, inv_q):
    i = pl.program_id(0)
    j = pl.program_id(1)
    nq = pl.num_programs(0)
    nk = pl.num_programs(1)

    am = am_ref[...]
    bm = bm_ref[...]

    @pl.when(i == 0)
    def _():
        bf = bm.astype(jnp.float32)
        b2_ref[:, pl.ds(j * _BK, _BK)] = jnp.sum(bf * bf, axis=0, keepdims=True)

    @pl.when(j == 0)
    def _():
        acc_ref[...] = jnp.full((_BQ, _CK), _BIG, jnp.float32)

    ms = []
    for c in range(_BK // _CK):
        g = jnp.dot(
            am, bm[:, c * _CK : (c + 1) * _CK], preferred_element_type=jnp.float32
        )
        b2 = b2_ref[:, pl.ds(j * _BK + c * _CK, _CK)]
        ms.append(b2 + g)
    while len(ms) > 1:
        ms = [jnp.minimum(ms[t], ms[t + 1]) for t in range(0, len(ms), 2)]
    acc_ref[...] = jnp.minimum(acc_ref[...], ms[0])

    @pl.when(j == nk - 1)
    def _():
        af = am.astype(jnp.float32)
        a2 = 0.25 * jnp.sum(af * af, axis=1, keepdims=True)
        d2 = a2 + jnp.min(acc_ref[...], axis=1, keepdims=True)
        psum = jnp.sum(jnp.sqrt(jnp.maximum(d2, 1e-12))).reshape(1, 1)
        tot = jnp.where(i == 0, psum, out_ref[...] + psum)
        out_ref[...] = jnp.where(i == nq - 1, tot * inv_q, tot)


@jax.jit
def kernel(target_embeddings, target_slice_idx, cluster_centers):
    del target_slice_idx
    q, d = target_embeddings.shape
    k = cluster_centers.shape[0]
    a_mm = (-2.0 * target_embeddings).astype(jnp.bfloat16)
    b_mm = cluster_centers.T.astype(jnp.bfloat16)

    out = pl.pallas_call(
        functools.partial(_nn_loss_kernel, inv_q=1.0 / q),
        grid=(q // _BQ, k // _BK),
        in_specs=[
            pl.BlockSpec((_BQ, d), lambda i, j: (i, 0)),
            pl.BlockSpec((d, _BK), lambda i, j: (0, j)),
        ],
        out_specs=pl.BlockSpec((1, 1), lambda i, j: (0, 0)),
        out_shape=jax.ShapeDtypeStruct((1, 1), jnp.float32),
        scratch_shapes=[
            pltpu.VMEM((_BQ, _CK), jnp.float32),
            pltpu.VMEM((1, k), jnp.float32),
        ],
    )(a_mm, b_mm)
    return out[0, 0]
